# baseline (speedup 1.0000x reference)
import jax
import jax.numpy as jnp
from jax.experimental import pallas as pl
from jax.experimental.pallas import tpu as pltpu

BB = 8


def _body(hour_ref, day_ref, ht_ref, dt_ref, x_ref, o_ref):
    pid = pl.program_id(0)
    T = hour_ref.shape[1]
    N = x_ref.shape[0] // BB
    for i in range(BB):
        b = pid * BB + i
        parts = []
        for t in range(T):
            h = hour_ref[b, t]
            d = day_ref[b, t]
            parts.append(ht_ref[pl.ds(h, 1), :])
            parts.append(dt_ref[pl.ds(d, 1), :])
        emb = jnp.concatenate(parts, axis=1)
        sl = pl.ds(i * N, N)
        o_ref[sl, :] = x_ref[sl, :] + emb


def kernel(x, hour, day, hour_table, day_table):
    B, N, T, D = x.shape
    TD = T * D
    x2 = x.reshape(B * N, TD)
    hour = hour.astype(jnp.int32)
    day = day.astype(jnp.int32)
    rows_per_step = BB * N

    grid_spec = pltpu.PrefetchScalarGridSpec(
        num_scalar_prefetch=2,
        grid=(B // BB,),
        in_specs=[
            pl.BlockSpec(hour_table.shape, lambda b, *_: (0, 0)),
            pl.BlockSpec(day_table.shape, lambda b, *_: (0, 0)),
            pl.BlockSpec((rows_per_step, TD), lambda b, *_: (b, 0)),
        ],
        out_specs=pl.BlockSpec((rows_per_step, TD), lambda b, *_: (b, 0)),
    )
    out = pl.pallas_call(
        _body,
        grid_spec=grid_spec,
        out_shape=jax.ShapeDtypeStruct((B * N, TD), x.dtype),
    )(hour, day, hour_table, day_table, x2)
    return out.reshape(B, N, T, D)


# speedup vs baseline: 2.5215x; 2.5215x over previous
import jax
import jax.numpy as jnp
from jax.experimental import pallas as pl
from jax.experimental.pallas import tpu as pltpu

NBUF = 4


def _in_copy(x_hbm, in_buf, in_sem, chunk, slot):
    return pltpu.make_async_copy(x_hbm.at[chunk], in_buf.at[slot],
                                 in_sem.at[slot])


def _out_copy(o_hbm, out_buf, out_sem, chunk, slot):
    return pltpu.make_async_copy(out_buf.at[slot], o_hbm.at[chunk],
                                 out_sem.at[slot])


def _body(hour_ref, day_ref, ht_ref, dt_ref, x_hbm, o_hbm,
          in_buf, out_buf, in_sem, out_sem):
    s = pl.program_id(0)
    nsteps = pl.num_programs(0)
    T = hour_ref.shape[1]
    slot = jax.lax.rem(s, NBUF)

    @pl.when(s == 0)
    def _prologue():
        for j in range(NBUF):
            _in_copy(x_hbm, in_buf, in_sem, j, j).start()

    _in_copy(x_hbm, in_buf, in_sem, s, slot).wait()

    @pl.when(s >= NBUF)
    def _drain():
        _out_copy(o_hbm, out_buf, out_sem, s - NBUF, slot).wait()

    parts = []
    for t in range(T):
        h = hour_ref[s, t]
        d = day_ref[s, t]
        parts.append(ht_ref[pl.ds(h, 1), :])
        parts.append(dt_ref[pl.ds(d, 1), :])
    emb = jnp.concatenate(parts, axis=1)
    out_buf[slot] = in_buf[slot] + emb

    _out_copy(o_hbm, out_buf, out_sem, s, slot).start()

    @pl.when(s + NBUF < nsteps)
    def _refill():
        _in_copy(x_hbm, in_buf, in_sem, s + NBUF, slot).start()

    @pl.when(s == nsteps - 1)
    def _epilogue():
        for j in range(1, NBUF + 1):
            c = nsteps - j
            _out_copy(o_hbm, out_buf, out_sem, c, jax.lax.rem(c, NBUF)).wait()


def kernel(x, hour, day, hour_table, day_table):
    B, N, T, D = x.shape
    TD = T * D
    x3 = x.reshape(B, N, TD)
    hour = hour.astype(jnp.int32)
    day = day.astype(jnp.int32)

    grid_spec = pltpu.PrefetchScalarGridSpec(
        num_scalar_prefetch=2,
        grid=(B,),
        in_specs=[
            pl.BlockSpec(hour_table.shape, lambda b, *_: (0, 0)),
            pl.BlockSpec(day_table.shape, lambda b, *_: (0, 0)),
            pl.BlockSpec(memory_space=pl.ANY),
        ],
        out_specs=pl.BlockSpec(memory_space=pl.ANY),
        scratch_shapes=[
            pltpu.VMEM((NBUF, N, TD), jnp.float32),
            pltpu.VMEM((NBUF, N, TD), jnp.float32),
            pltpu.SemaphoreType.DMA((NBUF,)),
            pltpu.SemaphoreType.DMA((NBUF,)),
        ],
    )
    out = pl.pallas_call(
        _body,
        grid_spec=grid_spec,
        out_shape=jax.ShapeDtypeStruct((B, N, TD), x.dtype),
    )(hour, day, hour_table, day_table, x3)
    return out.reshape(B, N, T, D)
